# baseline (speedup 1.0000x reference)
import functools

import jax
import jax.numpy as jnp
from jax import lax
from jax.experimental import pallas as pl
from jax.experimental.pallas import tpu as pltpu
from jax.experimental.pallas import tpu_sc as plsc

N = 10000
E = 160000
D = 128
DE = 16
MLP_H = 256
NUM_CLASSES = 32
G = 100
BN_EPS = 1e-5

NC = 2
NS = 16
NW = NC * NS
EPW = E // NW
TAIL = 8
NP = 10240
ROWS_PT = NP // NS


def _zero_vmem(ref, nrows, ncols):
  z = jnp.zeros((16,), jnp.float32)

  def body(i, _):
    for j in range(ncols // 16):
      ref[i, pl.ds(j * 16, 16)] = z
    return 0

  lax.fori_loop(0, nrows, body, 0)


CH = 128
NFULL = EPW // CH


CS = 64
NCH = EPW // CS
NBUF = 4
NQUAD = 18


def _sc_segsum_kernel():
  def body(x_hbm, src_hbm, dst_hbm, s_out, *refs):
    sidx_all, didx_all = refs[0], refs[1]
    rows = refs[2:6]
    didx_c = refs[6:10]
    sidx_t, didx_t = refs[10], refs[11]
    s_sh = refs[12]
    gsem = refs[13:17]
    ssem = refs[17:21]

    c = lax.axis_index("c")
    s = lax.axis_index("s")
    wid = c * NS + s
    base_w = wid * EPW
    row0 = s * ROWS_PT

    _zero_vmem(rows[0], CS, D)
    for j in range(ROWS_PT // CS):
      pltpu.sync_copy(rows[0], s_sh.at[pl.ds(row0 + j * CS, CS)])

    pltpu.sync_copy(src_hbm.at[pl.ds(base_w, EPW)], sidx_all)
    pltpu.sync_copy(dst_hbm.at[pl.ds(base_w, EPW)], didx_all)
    plsc.subcore_barrier()

    def g_start(k, b):
      pltpu.async_copy(x_hbm.at[sidx_all.at[pl.ds(k * CS, CS)]],
                       rows[b], gsem[b])

    def g_wait(k, b):
      pltpu.make_async_copy(x_hbm.at[sidx_all.at[pl.ds(k * CS, CS)]],
                            rows[b], gsem[b]).wait()

    def s_start(k, b):
      for j in range(CS // 16):
        didx_c[b][pl.ds(j * 16, 16)] = didx_all[pl.ds(k * CS + j * 16, 16)]
      pltpu.async_copy(rows[b], s_sh.at[didx_c[b]], ssem[b], add=True)

    def s_wait(b):
      pltpu.make_async_copy(rows[b], s_sh.at[didx_c[b]], ssem[b]).wait()

    dump = jnp.full((16,), N + 100, jnp.int32)
    for b in (2, 3):
      for j in range(CS // 16):
        didx_c[b][pl.ds(j * 16, 16)] = dump
      pltpu.async_copy(rows[b], s_sh.at[didx_c[b]], ssem[b], add=True)
    g_start(0, 0)
    g_start(1, 1)

    def quad(q, _):
      k0 = 4 * q
      for b in range(NBUF):
        k = k0 + b
        g_wait(k, b)
        s_start(k, b)
        s_wait((b + 2) % NBUF)
        g_start(k + 2, (b + 2) % NBUF)
      return 0

    lax.fori_loop(0, NQUAD, quad, 0)

    for k in range(4 * NQUAD, NCH):
      b = k % NBUF
      g_wait(k, b)
      s_start(k, b)
      if k + 2 < NCH:
        s_wait((b + 2) % NBUF)
        g_start(k + 2, (b + 2) % NBUF)
    for b in range(NBUF):
      s_wait(b)

    base = base_w + NCH * CS
    pltpu.sync_copy(src_hbm.at[pl.ds(base, TAIL)], sidx_t)
    pltpu.sync_copy(dst_hbm.at[pl.ds(base, TAIL)], didx_t)
    pltpu.async_copy(x_hbm.at[sidx_t], rows[0].at[pl.ds(0, TAIL)],
                     gsem[0]).wait()
    pltpu.sync_copy(rows[0].at[pl.ds(0, TAIL)], s_sh.at[didx_t], add=True)

    plsc.subcore_barrier()
    pltpu.sync_copy(s_sh.at[pl.ds(row0, ROWS_PT)],
                    s_out.at[c, pl.ds(row0, ROWS_PT)])

  mesh = plsc.VectorSubcoreMesh(core_axis_name="c", subcore_axis_name="s",
                                num_cores=NC, num_subcores=NS)
  return pl.kernel(
      body,
      out_type=jax.ShapeDtypeStruct((NC, NP, D), jnp.float32),
      mesh=mesh,
      scratch_types=(
          [pltpu.VMEM((EPW,), jnp.int32), pltpu.VMEM((EPW,), jnp.int32)]
          + [pltpu.VMEM((CS, D), jnp.float32)] * NBUF
          + [pltpu.VMEM((CS,), jnp.int32)] * NBUF
          + [pltpu.VMEM((TAIL,), jnp.int32), pltpu.VMEM((TAIL,), jnp.int32)]
          + [pltpu.VMEM_SHARED((NP, D), jnp.float32)]
          + [pltpu.SemaphoreType.DMA] * (2 * NBUF)
      ))


def _sc_aux_kernel():
  ca = 64
  naux = EPW // ca

  def body(dst_hbm, ea_hbm, acc_out,
           didx_all, didx_ca, didx_cb, comb_a, comb_b, eat_a, eat_b,
           didx_t, acc_sh, sem_a, sem_b, sem_sa, sem_sb):
    c = lax.axis_index("c")
    s = lax.axis_index("s")
    wid = c * NS + s
    base_w = wid * EPW
    row0 = s * ROWS_PT

    _zero_vmem(comb_a, ca, D)
    for j in range(ROWS_PT // ca):
      pltpu.sync_copy(comb_a, acc_sh.at[pl.ds(row0 + j * ca, ca)])
    one = jnp.ones((16,), jnp.float32)

    def fill1(comb):
      def go(i, _):
        comb[i, pl.ds(DE, 16)] = one
        return 0
      lax.fori_loop(0, ca, go, 0)

    fill1(comb_a)
    _zero_vmem(comb_b, ca, D)
    fill1(comb_b)
    pltpu.sync_copy(dst_hbm.at[pl.ds(base_w, EPW)], didx_all)
    plsc.subcore_barrier()

    def ea_start(k, eat, sem):
      pltpu.async_copy(ea_hbm.at[pl.ds(base_w + k * ca, ca)], eat, sem)

    def ea_wait(k, eat, sem):
      pltpu.make_async_copy(ea_hbm.at[pl.ds(base_w + k * ca, ca)],
                            eat, sem).wait()

    def retire(k, eat, comb, didx_c, sem):
      pltpu.make_async_copy(comb, acc_sh.at[didx_c], sem).wait()

      def cp(i, _):
        comb[i, pl.ds(0, DE)] = eat[i, pl.ds(0, DE)]
        return 0

      lax.fori_loop(0, ca, cp, 0)
      for j in range(ca // 16):
        didx_c[pl.ds(j * 16, 16)] = didx_all[pl.ds(k * ca + j * 16, 16)]
      pltpu.async_copy(comb, acc_sh.at[didx_c], sem, add=True)

    ea_start(0, eat_a, sem_a)
    dump = jnp.full((16,), N + 100, jnp.int32)
    for j in range(ca // 16):
      didx_ca[pl.ds(j * 16, 16)] = dump
      didx_cb[pl.ds(j * 16, 16)] = dump
    pltpu.async_copy(comb_a, acc_sh.at[didx_ca], sem_sa, add=True)
    pltpu.async_copy(comb_b, acc_sh.at[didx_cb], sem_sb, add=True)

    def pair(j, _):
      ka = 2 * j
      ea_start(ka + 1, eat_b, sem_b)
      ea_wait(ka, eat_a, sem_a)
      retire(ka, eat_a, comb_a, didx_ca, sem_sa)

      @pl.when(ka + 2 < naux)
      def _():
        ea_start(ka + 2, eat_a, sem_a)

      ea_wait(ka + 1, eat_b, sem_b)
      retire(ka + 1, eat_b, comb_b, didx_cb, sem_sb)
      return 0

    lax.fori_loop(0, naux // 2, pair, 0)

    pltpu.make_async_copy(comb_a, acc_sh.at[didx_ca], sem_sa).wait()
    pltpu.make_async_copy(comb_b, acc_sh.at[didx_cb], sem_sb).wait()
    base = base_w + naux * ca
    pltpu.sync_copy(dst_hbm.at[pl.ds(base, TAIL)], didx_t)
    pltpu.sync_copy(ea_hbm.at[pl.ds(base, TAIL)], eat_a.at[pl.ds(0, TAIL)])
    for i in range(TAIL):
      comb_a[i, pl.ds(0, DE)] = eat_a[i, pl.ds(0, DE)]
    pltpu.sync_copy(comb_a.at[pl.ds(0, TAIL)], acc_sh.at[didx_t], add=True)

    plsc.subcore_barrier()
    pltpu.sync_copy(acc_sh.at[pl.ds(row0, ROWS_PT)],
                    acc_out.at[c, pl.ds(row0, ROWS_PT)])

  mesh = plsc.VectorSubcoreMesh(core_axis_name="c", subcore_axis_name="s",
                                num_cores=NC, num_subcores=NS)
  return pl.kernel(
      body,
      out_type=jax.ShapeDtypeStruct((NC, NP, D), jnp.float32),
      mesh=mesh,
      scratch_types=[
          pltpu.VMEM((EPW,), jnp.int32),
          pltpu.VMEM((ca,), jnp.int32), pltpu.VMEM((ca,), jnp.int32),
          pltpu.VMEM((ca, D), jnp.float32), pltpu.VMEM((ca, D), jnp.float32),
          pltpu.VMEM((ca, DE), jnp.float32), pltpu.VMEM((ca, DE), jnp.float32),
          pltpu.VMEM((TAIL,), jnp.int32),
          pltpu.VMEM_SHARED((NP, D), jnp.float32),
          pltpu.SemaphoreType.DMA, pltpu.SemaphoreType.DMA,
          pltpu.SemaphoreType.DMA, pltpu.SemaphoreType.DMA,
      ])


BLK = 1000
GRID = N // BLK
BN_C = 1.0 / (1.0 + BN_EPS) ** 0.5


def _dense_math(xb, sp0, sp1, aux0, aux1, w_ref, b_ref, g_ref, be_ref):
  s_full = sp0 + sp1 + xb
  a_full = aux0[:, 0:DE] + aux1[:, 0:DE] + 1.0
  dg = aux0[:, DE:DE + 1] + aux1[:, DE:DE + 1] + 1.0
  wi = w_ref[0:D, :]
  wj = w_ref[D:2 * D, :]
  we = w_ref[2 * D:2 * D + DE, :]
  agg = (jnp.dot(xb * dg, wi, preferred_element_type=jnp.float32)
         + jnp.dot(s_full, wj, preferred_element_type=jnp.float32)
         + jnp.dot(a_full, we, preferred_element_type=jnp.float32)
         + dg * b_ref[...])
  h = jnp.maximum(agg, 0.0)
  h = h * (g_ref[...] * BN_C) + be_ref[...]
  return jnp.maximum(h, 0.0)


def _dense_body(h_ref, sp_ref, aux_ref, w_ref, b_ref, g_ref, be_ref,
                out_ref):
  out_ref[...] = _dense_math(h_ref[...], sp_ref[0], sp_ref[1], aux_ref[0],
                             aux_ref[1], w_ref, b_ref, g_ref, be_ref)


def _onehot(batch_blk):
  return jnp.equal(
      batch_blk,
      lax.broadcasted_iota(jnp.int32, (1, G), 1)).astype(jnp.float32)


def _fused2_body(h_ref, sp_ref, aux_ref, w_ref, b_ref, g_ref, be_ref,
                 batch_ref, fc1_ref, fc1b_ref, fc2_ref, fc2b_ref,
                 out_ref, h2_scr, ge_scr):
  p = pl.program_id(0)
  i = pl.program_id(1)
  m = _onehot(batch_ref[...])

  @pl.when(p == 0)
  def _():
    h2 = _dense_math(h_ref[...], sp_ref[0], sp_ref[1], aux_ref[0],
                     aux_ref[1], w_ref, b_ref, g_ref, be_ref)
    h2_scr[pl.ds(i * BLK, BLK), :] = h2

    @pl.when(i == 0)
    def _():
      ge_scr[...] = jnp.zeros_like(ge_scr)

    ge_scr[...] += lax.dot_general(m, h2, (((0,), (0,)), ((), ())),
                                   preferred_element_type=jnp.float32)
    out_ref[...] = jnp.zeros_like(out_ref)

  @pl.when(p == 1)
  def _():
    h2 = h2_scr[pl.ds(i * BLK, BLK), :]
    pp = jnp.dot(ge_scr[...], fc1_ref[D:2 * D, :],
                 preferred_element_type=jnp.float32)
    z = (jnp.dot(h2, fc1_ref[0:D, :], preferred_element_type=jnp.float32)
         + jnp.dot(m, pp, preferred_element_type=jnp.float32)
         + fc1b_ref[...])
    z = jnp.maximum(z, 0.0)
    out_ref[...] = (jnp.dot(z, fc2_ref[...],
                            preferred_element_type=jnp.float32)
                    + fc2b_ref[...])


def _full(shape):
  return pl.BlockSpec(shape, lambda i: (0,) * len(shape))


def _dense_specs():
  return [
      pl.BlockSpec((BLK, D), lambda i: (i, 0)),
      pl.BlockSpec((NC, BLK, D), lambda i: (0, i, 0)),
      pl.BlockSpec((NC, BLK, D), lambda i: (0, i, 0)),
      _full((2 * D + DE, D)),
      _full((1, D)),
      _full((1, D)),
      _full((1, D)),
  ]


def kernel(x, edge_index, edge_attr, batch, mask, W0, b0, g0, be0,
           W1, b1, g1, be1, fc1_w, fc1_b, fc2_w, fc2_b):
  del mask
  src = edge_index[0]
  dst = edge_index[1]
  batch2d = batch.reshape(N, 1)

  sc_s = _sc_segsum_kernel()
  sc_aux = _sc_aux_kernel()

  s0_p = sc_s(x, src, dst)
  aux_p = sc_aux(dst, edge_attr)

  dense1 = pl.pallas_call(
      _dense_body,
      grid=(GRID,),
      in_specs=_dense_specs(),
      out_specs=pl.BlockSpec((BLK, D), lambda i: (i, 0)),
      out_shape=jax.ShapeDtypeStruct((N, D), jnp.float32),
      compiler_params=pltpu.CompilerParams(
          dimension_semantics=("arbitrary",)),
  )
  h1 = dense1(x, s0_p, aux_p, W0, b0.reshape(1, D), g0.reshape(1, D),
              be0.reshape(1, D))

  s1_p = sc_s(h1, src, dst)

  fused = pl.pallas_call(
      _fused2_body,
      grid=(2, GRID),
      in_specs=[
          pl.BlockSpec((BLK, D), lambda p, i: (i, 0)),
          pl.BlockSpec((NC, BLK, D), lambda p, i: (0, i, 0)),
          pl.BlockSpec((NC, BLK, D), lambda p, i: (0, i, 0)),
          pl.BlockSpec((2 * D + DE, D), lambda p, i: (0, 0)),
          pl.BlockSpec((1, D), lambda p, i: (0, 0)),
          pl.BlockSpec((1, D), lambda p, i: (0, 0)),
          pl.BlockSpec((1, D), lambda p, i: (0, 0)),
          pl.BlockSpec((BLK, 1), lambda p, i: (i, 0)),
          pl.BlockSpec((2 * D, MLP_H), lambda p, i: (0, 0)),
          pl.BlockSpec((1, MLP_H), lambda p, i: (0, 0)),
          pl.BlockSpec((MLP_H, NUM_CLASSES), lambda p, i: (0, 0)),
          pl.BlockSpec((1, NUM_CLASSES), lambda p, i: (0, 0)),
      ],
      out_specs=pl.BlockSpec((BLK, NUM_CLASSES), lambda p, i: (i, 0)),
      out_shape=jax.ShapeDtypeStruct((N, NUM_CLASSES), jnp.float32),
      scratch_shapes=[pltpu.VMEM((N, D), jnp.float32),
                      pltpu.VMEM((G, D), jnp.float32)],
      compiler_params=pltpu.CompilerParams(
          dimension_semantics=("arbitrary", "arbitrary")),
  )
  out = fused(h1, s1_p, aux_p, W1, b1.reshape(1, D), g1.reshape(1, D),
              be1.reshape(1, D), batch2d, fc1_w, fc1_b.reshape(1, MLP_H),
              fc2_w, fc2_b.reshape(1, NUM_CLASSES))
  return out


# speedup vs baseline: 1.0120x; 1.0120x over previous
import functools

import jax
import jax.numpy as jnp
from jax import lax
from jax.experimental import pallas as pl
from jax.experimental.pallas import tpu as pltpu
from jax.experimental.pallas import tpu_sc as plsc

N = 10000
E = 160000
D = 128
DE = 16
MLP_H = 256
NUM_CLASSES = 32
G = 100
BN_EPS = 1e-5

NC = 2
NS = 16
NW = NC * NS
EPW = E // NW
TAIL = 8
NP = 10240
ROWS_PT = NP // NS


def _zero_vmem(ref, nrows, ncols):
  z = jnp.zeros((16,), jnp.float32)

  def body(i, _):
    for j in range(ncols // 16):
      ref[i, pl.ds(j * 16, 16)] = z
    return 0

  lax.fori_loop(0, nrows, body, 0)


CH = 128
NFULL = EPW // CH


CS = 64
NCH = EPW // CS
NBUF = 4
NQUAD = 18


def _sc_segsum_kernel():
  def body(x_hbm, src_hbm, dst_hbm, s_out, *refs):
    sidx_all, didx_all = refs[0], refs[1]
    rows = refs[2:6]
    didx_c = refs[6:10]
    sidx_t, didx_t = refs[10], refs[11]
    s_sh = refs[12]
    gsem = refs[13:17]
    ssem = refs[17:21]

    c = lax.axis_index("c")
    s = lax.axis_index("s")
    wid = c * NS + s
    base_w = wid * EPW
    row0 = s * ROWS_PT

    _zero_vmem(rows[0], CS, D)
    for j in range(ROWS_PT // CS):
      pltpu.sync_copy(rows[0], s_sh.at[pl.ds(row0 + j * CS, CS)])

    pltpu.sync_copy(src_hbm.at[pl.ds(base_w, EPW)], sidx_all)
    pltpu.sync_copy(dst_hbm.at[pl.ds(base_w, EPW)], didx_all)
    plsc.subcore_barrier()

    def g_start(k, b):
      pltpu.async_copy(x_hbm.at[sidx_all.at[pl.ds(k * CS, CS)]],
                       rows[b], gsem[b])

    def g_wait(k, b):
      pltpu.make_async_copy(x_hbm.at[sidx_all.at[pl.ds(k * CS, CS)]],
                            rows[b], gsem[b]).wait()

    def s_start(k, b):
      for j in range(CS // 16):
        didx_c[b][pl.ds(j * 16, 16)] = didx_all[pl.ds(k * CS + j * 16, 16)]
      pltpu.async_copy(rows[b], s_sh.at[didx_c[b]], ssem[b], add=True)

    def s_wait(b):
      pltpu.make_async_copy(rows[b], s_sh.at[didx_c[b]], ssem[b]).wait()

    dump = jnp.full((16,), N + 100, jnp.int32)
    for b in (2, 3):
      for j in range(CS // 16):
        didx_c[b][pl.ds(j * 16, 16)] = dump
      pltpu.async_copy(rows[b], s_sh.at[didx_c[b]], ssem[b], add=True)
    g_start(0, 0)
    g_start(1, 1)

    def quad(q, _):
      k0 = 4 * q
      for b in range(NBUF):
        k = k0 + b
        g_wait(k, b)
        s_start(k, b)
        s_wait((b + 2) % NBUF)
        g_start(k + 2, (b + 2) % NBUF)
      return 0

    lax.fori_loop(0, NQUAD, quad, 0)

    for k in range(4 * NQUAD, NCH):
      b = k % NBUF
      g_wait(k, b)
      s_start(k, b)
      if k + 2 < NCH:
        s_wait((b + 2) % NBUF)
        g_start(k + 2, (b + 2) % NBUF)
    for b in range(NBUF):
      s_wait(b)

    base = base_w + NCH * CS
    pltpu.sync_copy(src_hbm.at[pl.ds(base, TAIL)], sidx_t)
    pltpu.sync_copy(dst_hbm.at[pl.ds(base, TAIL)], didx_t)
    pltpu.async_copy(x_hbm.at[sidx_t], rows[0].at[pl.ds(0, TAIL)],
                     gsem[0]).wait()
    pltpu.sync_copy(rows[0].at[pl.ds(0, TAIL)], s_sh.at[didx_t], add=True)

    plsc.subcore_barrier()
    pltpu.sync_copy(s_sh.at[pl.ds(row0, ROWS_PT)],
                    s_out.at[c, pl.ds(row0, ROWS_PT)])

  mesh = plsc.VectorSubcoreMesh(core_axis_name="c", subcore_axis_name="s",
                                num_cores=NC, num_subcores=NS)
  return pl.kernel(
      body,
      out_type=jax.ShapeDtypeStruct((NC, NP, D), jnp.float32),
      mesh=mesh,
      scratch_types=(
          [pltpu.VMEM((EPW,), jnp.int32), pltpu.VMEM((EPW,), jnp.int32)]
          + [pltpu.VMEM((CS, D), jnp.float32)] * NBUF
          + [pltpu.VMEM((CS,), jnp.int32)] * NBUF
          + [pltpu.VMEM((TAIL,), jnp.int32), pltpu.VMEM((TAIL,), jnp.int32)]
          + [pltpu.VMEM_SHARED((NP, D), jnp.float32)]
          + [pltpu.SemaphoreType.DMA] * (2 * NBUF)
      ))


def _sc_aux_kernel():
  ca = 64
  naux = EPW // ca

  def body(dst_hbm, ea_hbm, acc_out,
           didx_all, didx_ca, didx_cb, comb_a, comb_b, eat_a, eat_b,
           didx_t, acc_sh, sem_a, sem_b, sem_sa, sem_sb):
    c = lax.axis_index("c")
    s = lax.axis_index("s")
    wid = c * NS + s
    base_w = wid * EPW
    row0 = s * ROWS_PT

    _zero_vmem(comb_a, ca, D)
    for j in range(ROWS_PT // ca):
      pltpu.sync_copy(comb_a, acc_sh.at[pl.ds(row0 + j * ca, ca)])
    one = jnp.ones((16,), jnp.float32)

    def fill1(comb):
      def go(i, _):
        comb[i, pl.ds(DE, 16)] = one
        return 0
      lax.fori_loop(0, ca, go, 0)

    fill1(comb_a)
    _zero_vmem(comb_b, ca, D)
    fill1(comb_b)
    pltpu.sync_copy(dst_hbm.at[pl.ds(base_w, EPW)], didx_all)
    plsc.subcore_barrier()

    def ea_start(k, eat, sem):
      pltpu.async_copy(ea_hbm.at[pl.ds(base_w + k * ca, ca)], eat, sem)

    def ea_wait(k, eat, sem):
      pltpu.make_async_copy(ea_hbm.at[pl.ds(base_w + k * ca, ca)],
                            eat, sem).wait()

    def retire(k, eat, comb, didx_c, sem):
      pltpu.make_async_copy(comb, acc_sh.at[didx_c], sem).wait()

      def cp(i, _):
        comb[i, pl.ds(0, DE)] = eat[i, pl.ds(0, DE)]
        return 0

      lax.fori_loop(0, ca, cp, 0)
      for j in range(ca // 16):
        didx_c[pl.ds(j * 16, 16)] = didx_all[pl.ds(k * ca + j * 16, 16)]
      pltpu.async_copy(comb, acc_sh.at[didx_c], sem, add=True)

    ea_start(0, eat_a, sem_a)
    dump = jnp.full((16,), N + 100, jnp.int32)
    for j in range(ca // 16):
      didx_ca[pl.ds(j * 16, 16)] = dump
      didx_cb[pl.ds(j * 16, 16)] = dump
    pltpu.async_copy(comb_a, acc_sh.at[didx_ca], sem_sa, add=True)
    pltpu.async_copy(comb_b, acc_sh.at[didx_cb], sem_sb, add=True)

    def pair(j, _):
      ka = 2 * j
      ea_start(ka + 1, eat_b, sem_b)
      ea_wait(ka, eat_a, sem_a)
      retire(ka, eat_a, comb_a, didx_ca, sem_sa)

      @pl.when(ka + 2 < naux)
      def _():
        ea_start(ka + 2, eat_a, sem_a)

      ea_wait(ka + 1, eat_b, sem_b)
      retire(ka + 1, eat_b, comb_b, didx_cb, sem_sb)
      return 0

    lax.fori_loop(0, naux // 2, pair, 0)

    pltpu.make_async_copy(comb_a, acc_sh.at[didx_ca], sem_sa).wait()
    pltpu.make_async_copy(comb_b, acc_sh.at[didx_cb], sem_sb).wait()
    base = base_w + naux * ca
    pltpu.sync_copy(dst_hbm.at[pl.ds(base, TAIL)], didx_t)
    pltpu.sync_copy(ea_hbm.at[pl.ds(base, TAIL)], eat_a.at[pl.ds(0, TAIL)])
    for i in range(TAIL):
      comb_a[i, pl.ds(0, DE)] = eat_a[i, pl.ds(0, DE)]
    pltpu.sync_copy(comb_a.at[pl.ds(0, TAIL)], acc_sh.at[didx_t], add=True)

    plsc.subcore_barrier()
    pltpu.sync_copy(acc_sh.at[pl.ds(row0, ROWS_PT)],
                    acc_out.at[c, pl.ds(row0, ROWS_PT)])

  mesh = plsc.VectorSubcoreMesh(core_axis_name="c", subcore_axis_name="s",
                                num_cores=NC, num_subcores=NS)
  return pl.kernel(
      body,
      out_type=jax.ShapeDtypeStruct((NC, NP, D), jnp.float32),
      mesh=mesh,
      scratch_types=[
          pltpu.VMEM((EPW,), jnp.int32),
          pltpu.VMEM((ca,), jnp.int32), pltpu.VMEM((ca,), jnp.int32),
          pltpu.VMEM((ca, D), jnp.float32), pltpu.VMEM((ca, D), jnp.float32),
          pltpu.VMEM((ca, DE), jnp.float32), pltpu.VMEM((ca, DE), jnp.float32),
          pltpu.VMEM((TAIL,), jnp.int32),
          pltpu.VMEM_SHARED((NP, D), jnp.float32),
          pltpu.SemaphoreType.DMA, pltpu.SemaphoreType.DMA,
          pltpu.SemaphoreType.DMA, pltpu.SemaphoreType.DMA,
      ])


BLK = 1000
GRID = N // BLK
BN_C = 1.0 / (1.0 + BN_EPS) ** 0.5


def _dense_math(xb, sp0, sp1, aux0, aux1, w_ref, b_ref, g_ref, be_ref):
  s_full = sp0 + sp1 + xb
  a_full = aux0[:, 0:DE] + aux1[:, 0:DE] + 1.0
  dg = aux0[:, DE:DE + 1] + aux1[:, DE:DE + 1] + 1.0
  wi = w_ref[0:D, :]
  wj = w_ref[D:2 * D, :]
  we = w_ref[2 * D:2 * D + DE, :]
  agg = (jnp.dot(xb * dg, wi, preferred_element_type=jnp.float32)
         + jnp.dot(s_full, wj, preferred_element_type=jnp.float32)
         + jnp.dot(a_full, we, preferred_element_type=jnp.float32)
         + dg * b_ref[...])
  h = jnp.maximum(agg, 0.0)
  h = h * (g_ref[...] * BN_C) + be_ref[...]
  return jnp.maximum(h, 0.0)


def _dense_body(h_ref, sp_ref, aux_ref, w_ref, b_ref, g_ref, be_ref,
                out_ref):
  out_ref[...] = _dense_math(h_ref[...], sp_ref[0], sp_ref[1], aux_ref[0],
                             aux_ref[1], w_ref, b_ref, g_ref, be_ref)


def _onehot(batch_blk):
  return jnp.equal(
      batch_blk,
      lax.broadcasted_iota(jnp.int32, (1, G), 1)).astype(jnp.float32)


def _fused2_body(h_ref, sp_ref, aux_ref, w_ref, b_ref, g_ref, be_ref,
                 batch_ref, fc1_ref, fc1b_ref, fc2_ref, fc2b_ref,
                 out_ref, h2_scr, ge_scr):
  p = pl.program_id(0)
  i = pl.program_id(1)
  m = _onehot(batch_ref[...])

  @pl.when(p == 0)
  def _():
    h2 = _dense_math(h_ref[...], sp_ref[0], sp_ref[1], aux_ref[0],
                     aux_ref[1], w_ref, b_ref, g_ref, be_ref)
    h2_scr[pl.ds(i * BLK, BLK), :] = h2

    @pl.when(i == 0)
    def _():
      ge_scr[...] = jnp.zeros_like(ge_scr)

    ge_scr[...] += lax.dot_general(m, h2, (((0,), (0,)), ((), ())),
                                   preferred_element_type=jnp.float32)
    out_ref[...] = jnp.zeros_like(out_ref)

  @pl.when(p == 1)
  def _():
    h2 = h2_scr[pl.ds(i * BLK, BLK), :]
    pp = jnp.dot(ge_scr[...], fc1_ref[D:2 * D, :],
                 preferred_element_type=jnp.float32)
    z = (jnp.dot(h2, fc1_ref[0:D, :], preferred_element_type=jnp.float32)
         + jnp.dot(m, pp, preferred_element_type=jnp.float32)
         + fc1b_ref[...])
    z = jnp.maximum(z, 0.0)
    out_ref[...] = (jnp.dot(z, fc2_ref[...],
                            preferred_element_type=jnp.float32)
                    + fc2b_ref[...])


def _full(shape):
  return pl.BlockSpec(shape, lambda i: (0,) * len(shape))


def _dense_specs():
  return [
      pl.BlockSpec((BLK, D), lambda i: (i, 0)),
      pl.BlockSpec((NC, BLK, D), lambda i: (0, i, 0)),
      pl.BlockSpec((NC, BLK, D), lambda i: (0, i, 0)),
      _full((2 * D + DE, D)),
      _full((1, D)),
      _full((1, D)),
      _full((1, D)),
  ]


def kernel(x, edge_index, edge_attr, batch, mask, W0, b0, g0, be0,
           W1, b1, g1, be1, fc1_w, fc1_b, fc2_w, fc2_b):
  del mask
  src = edge_index[0]
  dst = edge_index[1]
  batch2d = batch.reshape(N, 1)

  sc_s = _sc_segsum_kernel()
  sc_aux = _sc_aux_kernel()

  s0_p = sc_s(x, src, dst)
  aux_p = sc_aux(dst, edge_attr)

  dense1 = pl.pallas_call(
      _dense_body,
      grid=(GRID,),
      in_specs=_dense_specs(),
      out_specs=pl.BlockSpec((BLK, D), lambda i: (i, 0)),
      out_shape=jax.ShapeDtypeStruct((N, D), jnp.float32),
      compiler_params=pltpu.CompilerParams(
          dimension_semantics=("arbitrary",)),
  )
  h1 = dense1(x, s0_p, aux_p, W0, b0.reshape(1, D), g0.reshape(1, D),
              be0.reshape(1, D))

  s1_p = sc_s(h1, src, dst)

  fused = pl.pallas_call(
      _fused2_body,
      grid=(2, GRID),
      in_specs=[
          pl.BlockSpec((BLK, D), lambda p, i: ((1 - p) * i, 0)),
          pl.BlockSpec((NC, BLK, D), lambda p, i: (0, (1 - p) * i, 0)),
          pl.BlockSpec((NC, BLK, D), lambda p, i: (0, (1 - p) * i, 0)),
          pl.BlockSpec((2 * D + DE, D), lambda p, i: (0, 0)),
          pl.BlockSpec((1, D), lambda p, i: (0, 0)),
          pl.BlockSpec((1, D), lambda p, i: (0, 0)),
          pl.BlockSpec((1, D), lambda p, i: (0, 0)),
          pl.BlockSpec((BLK, 1), lambda p, i: (i, 0)),
          pl.BlockSpec((2 * D, MLP_H), lambda p, i: (0, 0)),
          pl.BlockSpec((1, MLP_H), lambda p, i: (0, 0)),
          pl.BlockSpec((MLP_H, NUM_CLASSES), lambda p, i: (0, 0)),
          pl.BlockSpec((1, NUM_CLASSES), lambda p, i: (0, 0)),
      ],
      out_specs=pl.BlockSpec((BLK, NUM_CLASSES), lambda p, i: (i, 0)),
      out_shape=jax.ShapeDtypeStruct((N, NUM_CLASSES), jnp.float32),
      scratch_shapes=[pltpu.VMEM((N, D), jnp.float32),
                      pltpu.VMEM((G, D), jnp.float32)],
      compiler_params=pltpu.CompilerParams(
          dimension_semantics=("arbitrary", "arbitrary")),
  )
  out = fused(h1, s1_p, aux_p, W1, b1.reshape(1, D), g1.reshape(1, D),
              be1.reshape(1, D), batch2d, fc1_w, fc1_b.reshape(1, MLP_H),
              fc2_w, fc2_b.reshape(1, NUM_CLASSES))
  return out


# speedup vs baseline: 1.0160x; 1.0040x over previous
import jax
import jax.numpy as jnp
from jax import lax
from jax.experimental import pallas as pl
from jax.experimental.pallas import tpu as pltpu
from jax.experimental.pallas import tpu_sc as plsc

N = 10000
E = 160000
D = 128
DE = 16
MLP_H = 256
NUM_CLASSES = 32
G = 100
BN_EPS = 1e-5

NC = 2
NS = 16
NW = NC * NS
EPW = E // NW
TAIL = 8
NP = 10240
ROWS_PT = NP // NS


def _zero_vmem(ref, nrows, ncols):
  z = jnp.zeros((16,), jnp.float32)

  def body(i, _):
    for j in range(ncols // 16):
      ref[i, pl.ds(j * 16, 16)] = z
    return 0

  lax.fori_loop(0, nrows, body, 0)


CH = 128
NFULL = EPW // CH


CS = 64
NCH = EPW // CS
NBUF = 4
NQUAD = 18


def _sc_segsum_kernel():
  def body(x_hbm, src_hbm, dst_hbm, s_out, *refs):
    sidx_all, didx_all = refs[0], refs[1]
    rows = refs[2:6]
    didx_c = refs[6:10]
    sidx_t, didx_t = refs[10], refs[11]
    s_sh = refs[12]
    gsem = refs[13:17]
    ssem = refs[17:21]

    c = lax.axis_index("c")
    s = lax.axis_index("s")
    wid = c * NS + s
    base_w = wid * EPW
    row0 = s * ROWS_PT

    _zero_vmem(rows[0], CS, D)
    for j in range(ROWS_PT // CS):
      pltpu.sync_copy(rows[0], s_sh.at[pl.ds(row0 + j * CS, CS)])

    pltpu.sync_copy(src_hbm.at[pl.ds(base_w, EPW)], sidx_all)
    pltpu.sync_copy(dst_hbm.at[pl.ds(base_w, EPW)], didx_all)
    plsc.subcore_barrier()

    def g_start(k, b):
      pltpu.async_copy(x_hbm.at[sidx_all.at[pl.ds(k * CS, CS)]],
                       rows[b], gsem[b])

    def g_wait(k, b):
      pltpu.make_async_copy(x_hbm.at[sidx_all.at[pl.ds(k * CS, CS)]],
                            rows[b], gsem[b]).wait()

    def s_start(k, b):
      for j in range(CS // 16):
        didx_c[b][pl.ds(j * 16, 16)] = didx_all[pl.ds(k * CS + j * 16, 16)]
      pltpu.async_copy(rows[b], s_sh.at[didx_c[b]], ssem[b], add=True)

    def s_wait(b):
      pltpu.make_async_copy(rows[b], s_sh.at[didx_c[b]], ssem[b]).wait()

    dump = jnp.full((16,), N + 100, jnp.int32)
    for b in (2, 3):
      for j in range(CS // 16):
        didx_c[b][pl.ds(j * 16, 16)] = dump
      pltpu.async_copy(rows[b], s_sh.at[didx_c[b]], ssem[b], add=True)
    g_start(0, 0)
    g_start(1, 1)

    def quad(q, _):
      k0 = 4 * q
      for b in range(NBUF):
        k = k0 + b
        g_wait(k, b)
        s_start(k, b)
        s_wait((b + 2) % NBUF)
        g_start(k + 2, (b + 2) % NBUF)
      return 0

    lax.fori_loop(0, NQUAD, quad, 0)

    for k in range(4 * NQUAD, NCH):
      b = k % NBUF
      g_wait(k, b)
      s_start(k, b)
      if k + 2 < NCH:
        s_wait((b + 2) % NBUF)
        g_start(k + 2, (b + 2) % NBUF)
    for b in range(NBUF):
      s_wait(b)

    base = base_w + NCH * CS
    pltpu.sync_copy(src_hbm.at[pl.ds(base, TAIL)], sidx_t)
    pltpu.sync_copy(dst_hbm.at[pl.ds(base, TAIL)], didx_t)
    pltpu.async_copy(x_hbm.at[sidx_t], rows[0].at[pl.ds(0, TAIL)],
                     gsem[0]).wait()
    pltpu.sync_copy(rows[0].at[pl.ds(0, TAIL)], s_sh.at[didx_t], add=True)

    plsc.subcore_barrier()
    pltpu.sync_copy(s_sh.at[pl.ds(row0, ROWS_PT)],
                    s_out.at[c, pl.ds(row0, ROWS_PT)])

  mesh = plsc.VectorSubcoreMesh(core_axis_name="c", subcore_axis_name="s",
                                num_cores=NC, num_subcores=NS)
  return pl.kernel(
      body,
      out_type=jax.ShapeDtypeStruct((NC, NP, D), jnp.float32),
      mesh=mesh,
      scratch_types=(
          [pltpu.VMEM((EPW,), jnp.int32), pltpu.VMEM((EPW,), jnp.int32)]
          + [pltpu.VMEM((CS, D), jnp.float32)] * NBUF
          + [pltpu.VMEM((CS,), jnp.int32)] * NBUF
          + [pltpu.VMEM((TAIL,), jnp.int32), pltpu.VMEM((TAIL,), jnp.int32)]
          + [pltpu.VMEM_SHARED((NP, D), jnp.float32)]
          + [pltpu.SemaphoreType.DMA] * (2 * NBUF)
      ))


def _sc_aux_kernel():
  ca = 64
  naux = EPW // ca

  def body(dst_hbm, ea_hbm, acc_out,
           didx_all, didx_ca, didx_cb, comb_a, comb_b, eat_a, eat_b,
           didx_t, acc_sh, sem_a, sem_b, sem_sa, sem_sb):
    c = lax.axis_index("c")
    s = lax.axis_index("s")
    wid = c * NS + s
    base_w = wid * EPW
    row0 = s * ROWS_PT

    _zero_vmem(comb_a, ca, D)
    for j in range(ROWS_PT // ca):
      pltpu.sync_copy(comb_a, acc_sh.at[pl.ds(row0 + j * ca, ca)])
    one = jnp.ones((16,), jnp.float32)

    def fill1(comb):
      def go(i, _):
        comb[i, pl.ds(DE, 16)] = one
        return 0
      lax.fori_loop(0, ca, go, 0)

    fill1(comb_a)
    _zero_vmem(comb_b, ca, D)
    fill1(comb_b)
    pltpu.sync_copy(dst_hbm.at[pl.ds(base_w, EPW)], didx_all)
    plsc.subcore_barrier()

    def ea_start(k, eat, sem):
      pltpu.async_copy(ea_hbm.at[pl.ds(base_w + k * ca, ca)], eat, sem)

    def ea_wait(k, eat, sem):
      pltpu.make_async_copy(ea_hbm.at[pl.ds(base_w + k * ca, ca)],
                            eat, sem).wait()

    def retire(k, eat, comb, didx_c, sem):
      pltpu.make_async_copy(comb, acc_sh.at[didx_c], sem).wait()

      def cp(i, _):
        comb[i, pl.ds(0, DE)] = eat[i, pl.ds(0, DE)]
        return 0

      lax.fori_loop(0, ca, cp, 0)
      for j in range(ca // 16):
        didx_c[pl.ds(j * 16, 16)] = didx_all[pl.ds(k * ca + j * 16, 16)]
      pltpu.async_copy(comb, acc_sh.at[didx_c], sem, add=True)

    ea_start(0, eat_a, sem_a)
    dump = jnp.full((16,), N + 100, jnp.int32)
    for j in range(ca // 16):
      didx_ca[pl.ds(j * 16, 16)] = dump
      didx_cb[pl.ds(j * 16, 16)] = dump
    pltpu.async_copy(comb_a, acc_sh.at[didx_ca], sem_sa, add=True)
    pltpu.async_copy(comb_b, acc_sh.at[didx_cb], sem_sb, add=True)

    def pair(j, _):
      ka = 2 * j
      ea_start(ka + 1, eat_b, sem_b)
      ea_wait(ka, eat_a, sem_a)
      retire(ka, eat_a, comb_a, didx_ca, sem_sa)

      @pl.when(ka + 2 < naux)
      def _():
        ea_start(ka + 2, eat_a, sem_a)

      ea_wait(ka + 1, eat_b, sem_b)
      retire(ka + 1, eat_b, comb_b, didx_cb, sem_sb)
      return 0

    lax.fori_loop(0, naux // 2, pair, 0)

    pltpu.make_async_copy(comb_a, acc_sh.at[didx_ca], sem_sa).wait()
    pltpu.make_async_copy(comb_b, acc_sh.at[didx_cb], sem_sb).wait()
    base = base_w + naux * ca
    pltpu.sync_copy(dst_hbm.at[pl.ds(base, TAIL)], didx_t)
    pltpu.sync_copy(ea_hbm.at[pl.ds(base, TAIL)], eat_a.at[pl.ds(0, TAIL)])
    for i in range(TAIL):
      comb_a[i, pl.ds(0, DE)] = eat_a[i, pl.ds(0, DE)]
    pltpu.sync_copy(comb_a.at[pl.ds(0, TAIL)], acc_sh.at[didx_t], add=True)

    plsc.subcore_barrier()
    pltpu.sync_copy(acc_sh.at[pl.ds(row0, ROWS_PT)],
                    acc_out.at[c, pl.ds(row0, ROWS_PT)])

  mesh = plsc.VectorSubcoreMesh(core_axis_name="c", subcore_axis_name="s",
                                num_cores=NC, num_subcores=NS)
  return pl.kernel(
      body,
      out_type=jax.ShapeDtypeStruct((NC, NP, D), jnp.float32),
      mesh=mesh,
      scratch_types=[
          pltpu.VMEM((EPW,), jnp.int32),
          pltpu.VMEM((ca,), jnp.int32), pltpu.VMEM((ca,), jnp.int32),
          pltpu.VMEM((ca, D), jnp.float32), pltpu.VMEM((ca, D), jnp.float32),
          pltpu.VMEM((ca, DE), jnp.float32), pltpu.VMEM((ca, DE), jnp.float32),
          pltpu.VMEM((TAIL,), jnp.int32),
          pltpu.VMEM_SHARED((NP, D), jnp.float32),
          pltpu.SemaphoreType.DMA, pltpu.SemaphoreType.DMA,
          pltpu.SemaphoreType.DMA, pltpu.SemaphoreType.DMA,
      ])


BLK = 1000
GRID = N // BLK
BN_C = 1.0 / (1.0 + BN_EPS) ** 0.5


def _dense_math(xb, sp0, sp1, aux0, aux1, w_ref, b_ref, g_ref, be_ref):
  s_full = sp0 + sp1 + xb
  a_full = aux0[:, 0:DE] + aux1[:, 0:DE] + 1.0
  dg = aux0[:, DE:DE + 1] + aux1[:, DE:DE + 1] + 1.0
  wi = w_ref[0:D, :]
  wj = w_ref[D:2 * D, :]
  we = w_ref[2 * D:2 * D + DE, :]
  agg = (jnp.dot(xb * dg, wi, preferred_element_type=jnp.float32)
         + jnp.dot(s_full, wj, preferred_element_type=jnp.float32)
         + jnp.dot(a_full, we, preferred_element_type=jnp.float32)
         + dg * b_ref[...])
  h = jnp.maximum(agg, 0.0)
  h = h * (g_ref[...] * BN_C) + be_ref[...]
  return jnp.maximum(h, 0.0)


def _dense_body(h_ref, sp_ref, aux_ref, w_ref, b_ref, g_ref, be_ref,
                out_ref):
  out_ref[...] = _dense_math(h_ref[...], sp_ref[0], sp_ref[1], aux_ref[0],
                             aux_ref[1], w_ref, b_ref, g_ref, be_ref)


def _onehot(batch_blk):
  return jnp.equal(
      batch_blk,
      lax.broadcasted_iota(jnp.int32, (1, G), 1)).astype(jnp.float32)


def _dense2_body(h_ref, sp_ref, aux_ref, w_ref, b_ref, g_ref, be_ref,
                 batch_ref, out_ref, ge_ref):
  _dense_body(h_ref, sp_ref, aux_ref, w_ref, b_ref, g_ref, be_ref, out_ref)
  i = pl.program_id(0)
  m = _onehot(batch_ref[...])

  @pl.when(i == 0)
  def _():
    ge_ref[...] = jnp.zeros_like(ge_ref)

  ge_ref[...] += lax.dot_general(m, out_ref[...], (((0,), (0,)), ((), ())),
                                 preferred_element_type=jnp.float32)


def _mlp_body(h_ref, ge_ref, batch_ref, fc1_ref, fc1b_ref, fc2_ref, fc2b_ref,
              out_ref):
  m = _onehot(batch_ref[...])
  p = jnp.dot(ge_ref[...], fc1_ref[D:2 * D, :],
              preferred_element_type=jnp.float32)
  z = (jnp.dot(h_ref[...], fc1_ref[0:D, :],
               preferred_element_type=jnp.float32)
       + jnp.dot(m, p, preferred_element_type=jnp.float32)
       + fc1b_ref[...])
  z = jnp.maximum(z, 0.0)
  out_ref[...] = (jnp.dot(z, fc2_ref[...], preferred_element_type=jnp.float32)
                  + fc2b_ref[...])


def _full(shape):
  return pl.BlockSpec(shape, lambda i: (0,) * len(shape))


def _dense_specs():
  return [
      pl.BlockSpec((BLK, D), lambda i: (i, 0)),
      pl.BlockSpec((NC, BLK, D), lambda i: (0, i, 0)),
      pl.BlockSpec((NC, BLK, D), lambda i: (0, i, 0)),
      _full((2 * D + DE, D)),
      _full((1, D)),
      _full((1, D)),
      _full((1, D)),
  ]


def kernel(x, edge_index, edge_attr, batch, mask, W0, b0, g0, be0,
           W1, b1, g1, be1, fc1_w, fc1_b, fc2_w, fc2_b):
  del mask
  src = edge_index[0]
  dst = edge_index[1]
  batch2d = batch.reshape(N, 1)

  sc_s = _sc_segsum_kernel()
  sc_aux = _sc_aux_kernel()

  s0_p = sc_s(x, src, dst)
  aux_p = sc_aux(dst, edge_attr)

  dense1 = pl.pallas_call(
      _dense_body,
      grid=(GRID,),
      in_specs=_dense_specs(),
      out_specs=pl.BlockSpec((BLK, D), lambda i: (i, 0)),
      out_shape=jax.ShapeDtypeStruct((N, D), jnp.float32),
      compiler_params=pltpu.CompilerParams(
          dimension_semantics=("arbitrary",)),
  )
  h1 = dense1(x, s0_p, aux_p, W0, b0.reshape(1, D), g0.reshape(1, D),
              be0.reshape(1, D))

  s1_p = sc_s(h1, src, dst)

  dense2 = pl.pallas_call(
      _dense2_body,
      grid=(GRID,),
      in_specs=_dense_specs() + [pl.BlockSpec((BLK, 1), lambda i: (i, 0))],
      out_specs=[pl.BlockSpec((BLK, D), lambda i: (i, 0)),
                 _full((G, D))],
      out_shape=[jax.ShapeDtypeStruct((N, D), jnp.float32),
                 jax.ShapeDtypeStruct((G, D), jnp.float32)],
      compiler_params=pltpu.CompilerParams(
          dimension_semantics=("arbitrary",)),
  )
  h2, ge = dense2(h1, s1_p, aux_p, W1, b1.reshape(1, D),
                  g1.reshape(1, D), be1.reshape(1, D), batch2d)

  mlp = pl.pallas_call(
      _mlp_body,
      grid=(GRID,),
      in_specs=[
          pl.BlockSpec((BLK, D), lambda i: (i, 0)),
          _full((G, D)),
          pl.BlockSpec((BLK, 1), lambda i: (i, 0)),
          _full((2 * D, MLP_H)),
          _full((1, MLP_H)),
          _full((MLP_H, NUM_CLASSES)),
          _full((1, NUM_CLASSES)),
      ],
      out_specs=pl.BlockSpec((BLK, NUM_CLASSES), lambda i: (i, 0)),
      out_shape=jax.ShapeDtypeStruct((N, NUM_CLASSES), jnp.float32),
      compiler_params=pltpu.CompilerParams(
          dimension_semantics=("arbitrary",)),
  )
  out = mlp(h2, ge, batch2d, fc1_w, fc1_b.reshape(1, MLP_H), fc2_w,
            fc2_b.reshape(1, NUM_CLASSES))
  return out


# speedup vs baseline: 1.0520x; 1.0354x over previous
import jax
import jax.numpy as jnp
from jax import lax
from jax.experimental import pallas as pl
from jax.experimental.pallas import tpu as pltpu
from jax.experimental.pallas import tpu_sc as plsc

N = 10000
E = 160000
D = 128
DE = 16
MLP_H = 256
NUM_CLASSES = 32
G = 100
BN_EPS = 1e-5

NC = 2
NS = 16
NW = NC * NS
EPW = E // NW
TAIL = 8
NP = 10240
ROWS_PT = NP // NS


def _zero_vmem(ref, nrows, ncols):
  z = jnp.zeros((16,), jnp.float32)

  def body(i, _):
    for j in range(ncols // 16):
      ref[i, pl.ds(j * 16, 16)] = z
    return 0

  lax.fori_loop(0, nrows, body, 0)


CH = 128
NFULL = EPW // CH


CS = 64
NCH = EPW // CS
NBUF = 4
NQUAD = 18


def _sc_segsum_kernel():
  def body(x_hbm, src_hbm, dst_hbm, s_out, *refs):
    sidx_all, didx_all = refs[0], refs[1]
    rows = refs[2:6]
    didx_c = refs[6:10]
    sidx_t, didx_t = refs[10], refs[11]
    s_sh = refs[12]
    gsem = refs[13:17]
    ssem = refs[17:21]

    c = lax.axis_index("c")
    s = lax.axis_index("s")
    wid = c * NS + s
    base_w = wid * EPW
    row0 = s * ROWS_PT

    _zero_vmem(rows[0], CS, D)
    for j in range(ROWS_PT // CS):
      pltpu.sync_copy(rows[0], s_sh.at[pl.ds(row0 + j * CS, CS)])

    pltpu.sync_copy(src_hbm.at[pl.ds(base_w, EPW)], sidx_all)
    pltpu.sync_copy(dst_hbm.at[pl.ds(base_w, EPW)], didx_all)
    plsc.subcore_barrier()

    def g_start(k, b):
      pltpu.async_copy(x_hbm.at[sidx_all.at[pl.ds(k * CS, CS)]],
                       rows[b], gsem[b])

    def g_wait(k, b):
      pltpu.make_async_copy(x_hbm.at[sidx_all.at[pl.ds(k * CS, CS)]],
                            rows[b], gsem[b]).wait()

    def s_start(k, b):
      for j in range(CS // 16):
        didx_c[b][pl.ds(j * 16, 16)] = didx_all[pl.ds(k * CS + j * 16, 16)]
      pltpu.async_copy(rows[b], s_sh.at[didx_c[b]], ssem[b], add=True)

    def s_wait(b):
      pltpu.make_async_copy(rows[b], s_sh.at[didx_c[b]], ssem[b]).wait()

    dump = jnp.full((16,), N + 100, jnp.int32)
    for b in (2, 3):
      for j in range(CS // 16):
        didx_c[b][pl.ds(j * 16, 16)] = dump
      pltpu.async_copy(rows[b], s_sh.at[didx_c[b]], ssem[b], add=True)
    g_start(0, 0)
    g_start(1, 1)

    def quad(q, _):
      k0 = 4 * q
      for b in range(NBUF):
        k = k0 + b
        g_wait(k, b)
        s_start(k, b)
        s_wait((b + 2) % NBUF)
        g_start(k + 2, (b + 2) % NBUF)
      return 0

    lax.fori_loop(0, NQUAD, quad, 0)

    for k in range(4 * NQUAD, NCH):
      b = k % NBUF
      g_wait(k, b)
      s_start(k, b)
      if k + 2 < NCH:
        s_wait((b + 2) % NBUF)
        g_start(k + 2, (b + 2) % NBUF)
    for b in range(NBUF):
      s_wait(b)

    base = base_w + NCH * CS
    pltpu.sync_copy(src_hbm.at[pl.ds(base, TAIL)], sidx_t)
    pltpu.sync_copy(dst_hbm.at[pl.ds(base, TAIL)], didx_t)
    pltpu.async_copy(x_hbm.at[sidx_t], rows[0].at[pl.ds(0, TAIL)],
                     gsem[0]).wait()
    pltpu.sync_copy(rows[0].at[pl.ds(0, TAIL)], s_sh.at[didx_t], add=True)

    plsc.subcore_barrier()
    pltpu.sync_copy(s_sh.at[pl.ds(row0, ROWS_PT)],
                    s_out.at[c, pl.ds(row0, ROWS_PT)])

  mesh = plsc.VectorSubcoreMesh(core_axis_name="c", subcore_axis_name="s",
                                num_cores=NC, num_subcores=NS)
  return pl.kernel(
      body,
      out_type=jax.ShapeDtypeStruct((NC, NP, D), jnp.float32),
      mesh=mesh,
      scratch_types=(
          [pltpu.VMEM((EPW,), jnp.int32), pltpu.VMEM((EPW,), jnp.int32)]
          + [pltpu.VMEM((CS, D), jnp.float32)] * NBUF
          + [pltpu.VMEM((CS,), jnp.int32)] * NBUF
          + [pltpu.VMEM((TAIL,), jnp.int32), pltpu.VMEM((TAIL,), jnp.int32)]
          + [pltpu.VMEM_SHARED((NP, D), jnp.float32)]
          + [pltpu.SemaphoreType.DMA] * (2 * NBUF)
      ))


def _sc_aux_kernel():
  ca = 64
  naux = EPW // ca

  def body(dst_hbm, ea_hbm, acc_out,
           didx_all, didx_ca, didx_cb, comb_a, comb_b, eat_a, eat_b,
           didx_t, acc_sh, sem_a, sem_b, sem_sa, sem_sb):
    c = lax.axis_index("c")
    s = lax.axis_index("s")
    wid = c * NS + s
    base_w = wid * EPW
    row0 = s * ROWS_PT

    _zero_vmem(comb_a, ca, D)
    for j in range(ROWS_PT // ca):
      pltpu.sync_copy(comb_a, acc_sh.at[pl.ds(row0 + j * ca, ca)])
    one = jnp.ones((16,), jnp.float32)

    def fill1(comb):
      def go(i, _):
        comb[i, pl.ds(DE, 16)] = one
        return 0
      lax.fori_loop(0, ca, go, 0)

    fill1(comb_a)
    _zero_vmem(comb_b, ca, D)
    fill1(comb_b)
    pltpu.sync_copy(dst_hbm.at[pl.ds(base_w, EPW)], didx_all)
    plsc.subcore_barrier()

    def ea_start(k, eat, sem):
      pltpu.async_copy(ea_hbm.at[pl.ds(base_w + k * ca, ca)], eat, sem)

    def ea_wait(k, eat, sem):
      pltpu.make_async_copy(ea_hbm.at[pl.ds(base_w + k * ca, ca)],
                            eat, sem).wait()

    def retire(k, eat, comb, didx_c, sem):
      pltpu.make_async_copy(comb, acc_sh.at[didx_c], sem).wait()
      for i in range(ca):
        comb[i, pl.ds(0, DE)] = eat[i, pl.ds(0, DE)]
      for j in range(ca // 16):
        didx_c[pl.ds(j * 16, 16)] = didx_all[pl.ds(k * ca + j * 16, 16)]
      pltpu.async_copy(comb, acc_sh.at[didx_c], sem, add=True)

    ea_start(0, eat_a, sem_a)
    dump = jnp.full((16,), N + 100, jnp.int32)
    for j in range(ca // 16):
      didx_ca[pl.ds(j * 16, 16)] = dump
      didx_cb[pl.ds(j * 16, 16)] = dump
    pltpu.async_copy(comb_a, acc_sh.at[didx_ca], sem_sa, add=True)
    pltpu.async_copy(comb_b, acc_sh.at[didx_cb], sem_sb, add=True)

    def pair(j, _):
      ka = 2 * j
      ea_start(ka + 1, eat_b, sem_b)
      ea_wait(ka, eat_a, sem_a)
      retire(ka, eat_a, comb_a, didx_ca, sem_sa)

      @pl.when(ka + 2 < naux)
      def _():
        ea_start(ka + 2, eat_a, sem_a)

      ea_wait(ka + 1, eat_b, sem_b)
      retire(ka + 1, eat_b, comb_b, didx_cb, sem_sb)
      return 0

    lax.fori_loop(0, naux // 2, pair, 0)

    pltpu.make_async_copy(comb_a, acc_sh.at[didx_ca], sem_sa).wait()
    pltpu.make_async_copy(comb_b, acc_sh.at[didx_cb], sem_sb).wait()
    base = base_w + naux * ca
    pltpu.sync_copy(dst_hbm.at[pl.ds(base, TAIL)], didx_t)
    pltpu.sync_copy(ea_hbm.at[pl.ds(base, TAIL)], eat_a.at[pl.ds(0, TAIL)])
    for i in range(TAIL):
      comb_a[i, pl.ds(0, DE)] = eat_a[i, pl.ds(0, DE)]
    pltpu.sync_copy(comb_a.at[pl.ds(0, TAIL)], acc_sh.at[didx_t], add=True)

    plsc.subcore_barrier()
    pltpu.sync_copy(acc_sh.at[pl.ds(row0, ROWS_PT)],
                    acc_out.at[c, pl.ds(row0, ROWS_PT)])

  mesh = plsc.VectorSubcoreMesh(core_axis_name="c", subcore_axis_name="s",
                                num_cores=NC, num_subcores=NS)
  return pl.kernel(
      body,
      out_type=jax.ShapeDtypeStruct((NC, NP, D), jnp.float32),
      mesh=mesh,
      scratch_types=[
          pltpu.VMEM((EPW,), jnp.int32),
          pltpu.VMEM((ca,), jnp.int32), pltpu.VMEM((ca,), jnp.int32),
          pltpu.VMEM((ca, D), jnp.float32), pltpu.VMEM((ca, D), jnp.float32),
          pltpu.VMEM((ca, DE), jnp.float32), pltpu.VMEM((ca, DE), jnp.float32),
          pltpu.VMEM((TAIL,), jnp.int32),
          pltpu.VMEM_SHARED((NP, D), jnp.float32),
          pltpu.SemaphoreType.DMA, pltpu.SemaphoreType.DMA,
          pltpu.SemaphoreType.DMA, pltpu.SemaphoreType.DMA,
      ])


BLK = 2000
GRID = N // BLK
BN_C = 1.0 / (1.0 + BN_EPS) ** 0.5


def _dense_math(xb, sp0, sp1, aux0, aux1, w_ref, b_ref, g_ref, be_ref):
  s_full = sp0 + sp1 + xb
  a_full = aux0[:, 0:DE] + aux1[:, 0:DE] + 1.0
  dg = aux0[:, DE:DE + 1] + aux1[:, DE:DE + 1] + 1.0
  wi = w_ref[0:D, :]
  wj = w_ref[D:2 * D, :]
  we = w_ref[2 * D:2 * D + DE, :]
  agg = (jnp.dot(xb * dg, wi, preferred_element_type=jnp.float32)
         + jnp.dot(s_full, wj, preferred_element_type=jnp.float32)
         + jnp.dot(a_full, we, preferred_element_type=jnp.float32)
         + dg * b_ref[...])
  h = jnp.maximum(agg, 0.0)
  h = h * (g_ref[...] * BN_C) + be_ref[...]
  return jnp.maximum(h, 0.0)


def _dense_body(h_ref, sp_ref, aux_ref, w_ref, b_ref, g_ref, be_ref,
                out_ref):
  out_ref[...] = _dense_math(h_ref[...], sp_ref[0], sp_ref[1], aux_ref[0],
                             aux_ref[1], w_ref, b_ref, g_ref, be_ref)


def _onehot(batch_blk):
  return jnp.equal(
      batch_blk,
      lax.broadcasted_iota(jnp.int32, (1, G), 1)).astype(jnp.float32)


def _dense2_body(h_ref, sp_ref, aux_ref, w_ref, b_ref, g_ref, be_ref,
                 batch_ref, out_ref, ge_ref):
  _dense_body(h_ref, sp_ref, aux_ref, w_ref, b_ref, g_ref, be_ref, out_ref)
  i = pl.program_id(0)
  m = _onehot(batch_ref[...])

  @pl.when(i == 0)
  def _():
    ge_ref[...] = jnp.zeros_like(ge_ref)

  ge_ref[...] += lax.dot_general(m, out_ref[...], (((0,), (0,)), ((), ())),
                                 preferred_element_type=jnp.float32)


def _mlp_body(h_ref, ge_ref, batch_ref, fc1_ref, fc1b_ref, fc2_ref, fc2b_ref,
              out_ref):
  m = _onehot(batch_ref[...])
  p = jnp.dot(ge_ref[...], fc1_ref[D:2 * D, :],
              preferred_element_type=jnp.float32)
  z = (jnp.dot(h_ref[...], fc1_ref[0:D, :],
               preferred_element_type=jnp.float32)
       + jnp.dot(m, p, preferred_element_type=jnp.float32)
       + fc1b_ref[...])
  z = jnp.maximum(z, 0.0)
  out_ref[...] = (jnp.dot(z, fc2_ref[...], preferred_element_type=jnp.float32)
                  + fc2b_ref[...])


def _full(shape):
  return pl.BlockSpec(shape, lambda i: (0,) * len(shape))


def _dense_specs():
  return [
      pl.BlockSpec((BLK, D), lambda i: (i, 0)),
      pl.BlockSpec((NC, BLK, D), lambda i: (0, i, 0)),
      pl.BlockSpec((NC, BLK, D), lambda i: (0, i, 0)),
      _full((2 * D + DE, D)),
      _full((1, D)),
      _full((1, D)),
      _full((1, D)),
  ]


def kernel(x, edge_index, edge_attr, batch, mask, W0, b0, g0, be0,
           W1, b1, g1, be1, fc1_w, fc1_b, fc2_w, fc2_b):
  del mask
  src = edge_index[0]
  dst = edge_index[1]
  batch2d = batch.reshape(N, 1)

  sc_s = _sc_segsum_kernel()
  sc_aux = _sc_aux_kernel()

  s0_p = sc_s(x, src, dst)
  aux_p = sc_aux(dst, edge_attr)

  dense1 = pl.pallas_call(
      _dense_body,
      grid=(GRID,),
      in_specs=_dense_specs(),
      out_specs=pl.BlockSpec((BLK, D), lambda i: (i, 0)),
      out_shape=jax.ShapeDtypeStruct((N, D), jnp.float32),
      compiler_params=pltpu.CompilerParams(
          dimension_semantics=("arbitrary",)),
  )
  h1 = dense1(x, s0_p, aux_p, W0, b0.reshape(1, D), g0.reshape(1, D),
              be0.reshape(1, D))

  s1_p = sc_s(h1, src, dst)

  dense2 = pl.pallas_call(
      _dense2_body,
      grid=(GRID,),
      in_specs=_dense_specs() + [pl.BlockSpec((BLK, 1), lambda i: (i, 0))],
      out_specs=[pl.BlockSpec((BLK, D), lambda i: (i, 0)),
                 _full((G, D))],
      out_shape=[jax.ShapeDtypeStruct((N, D), jnp.float32),
                 jax.ShapeDtypeStruct((G, D), jnp.float32)],
      compiler_params=pltpu.CompilerParams(
          dimension_semantics=("arbitrary",)),
  )
  h2, ge = dense2(h1, s1_p, aux_p, W1, b1.reshape(1, D),
                  g1.reshape(1, D), be1.reshape(1, D), batch2d)

  mlp = pl.pallas_call(
      _mlp_body,
      grid=(GRID,),
      in_specs=[
          pl.BlockSpec((BLK, D), lambda i: (i, 0)),
          _full((G, D)),
          pl.BlockSpec((BLK, 1), lambda i: (i, 0)),
          _full((2 * D, MLP_H)),
          _full((1, MLP_H)),
          _full((MLP_H, NUM_CLASSES)),
          _full((1, NUM_CLASSES)),
      ],
      out_specs=pl.BlockSpec((BLK, NUM_CLASSES), lambda i: (i, 0)),
      out_shape=jax.ShapeDtypeStruct((N, NUM_CLASSES), jnp.float32),
      compiler_params=pltpu.CompilerParams(
          dimension_semantics=("arbitrary",)),
  )
  out = mlp(h2, ge, batch2d, fc1_w, fc1_b.reshape(1, MLP_H), fc2_w,
            fc2_b.reshape(1, NUM_CLASSES))
  return out
